# baseline (speedup 1.0000x reference)
import functools

import jax
import jax.numpy as jnp
from jax import lax
from jax.experimental import pallas as pl
from jax.experimental.pallas import tpu as pltpu
from jax.experimental.pallas import tpu_sc as plsc

PD = 2
LD = 128
N = 10000
E = 320000
MP_TIMES = 2
LAYER_NUM = 2

NW = 32
CH = 128
EPW_CH = 80
EPW = CH * EPW_CH
EP = EPW * NW
NACC = 10240
RPT = NACC // 16
LN_EPS = 1e-5


def _gather_body(a_hbm, b_hbm, src_hbm, dst_hbm, ha_hbm, hb_hbm,
                 is_v, id_v, ra_v, rb_v,
                 sga0, sga1, sgb0, sgb1, swa0, swa1, swb0, swb1):
    wid = lax.axis_index("s") * 2 + lax.axis_index("c")
    base = wid * EPW
    sga = (sga0, sga1)
    sgb = (sgb0, sgb1)
    swa = (swa0, swa1)
    swb = (swb0, swb1)

    pltpu.sync_copy(src_hbm.at[pl.ds(wid * EPW_CH, EPW_CH)], is_v)
    pltpu.sync_copy(dst_hbm.at[pl.ds(wid * EPW_CH, EPW_CH)], id_v)

    def issue(j, b):
        pltpu.async_copy(a_hbm.at[is_v.at[j]], ra_v.at[b], sga[b])
        pltpu.async_copy(b_hbm.at[id_v.at[j]], rb_v.at[b], sgb[b])

    issue(0, 0)
    issue(1, 1)

    def pair(jj, carry):
        for b in range(2):
            j = jj * 2 + b
            off = base + j * CH
            pltpu.make_async_copy(a_hbm.at[is_v.at[j]], ra_v.at[b], sga[b]).wait()
            pltpu.make_async_copy(b_hbm.at[id_v.at[j]], rb_v.at[b], sgb[b]).wait()
            ca = pltpu.async_copy(ra_v.at[b], ha_hbm.at[pl.ds(off, CH)], swa[b])
            cb = pltpu.async_copy(rb_v.at[b], hb_hbm.at[pl.ds(off, CH)], swb[b])
            ca.wait()
            cb.wait()

            @pl.when(j + 2 < EPW_CH)
            def _():
                issue(j + 2, b)
        return carry

    lax.fori_loop(0, EPW_CH // 2, pair, 0)


def _sc_gather(a, b, srcp, dstp):
    mesh = plsc.VectorSubcoreMesh(core_axis_name="c", subcore_axis_name="s")
    f = pl.kernel(
        _gather_body,
        out_type=(jax.ShapeDtypeStruct((EP, LD), jnp.float32),
                  jax.ShapeDtypeStruct((EP, LD), jnp.float32)),
        mesh=mesh,
        scratch_types=[
            pltpu.VMEM((EPW_CH, CH), jnp.int32),
            pltpu.VMEM((EPW_CH, CH), jnp.int32),
            pltpu.VMEM((2, CH, LD), jnp.float32),
            pltpu.VMEM((2, CH, LD), jnp.float32),
        ] + [pltpu.SemaphoreType.DMA] * 8,
    )
    return f(a, b, srcp, dstp)


def _scatter_body(msg_hbm, dst_hbm, zer_hbm, out_hbm,
                  idx_v, m_v, acc_sh, sg0, sg1, ss0, ss1):
    c = lax.axis_index("c")
    s = lax.axis_index("s")
    wid = s * 2 + c
    base = wid * EPW
    sg = (sg0, sg1)
    ss = (ss0, ss1)

    pltpu.sync_copy(zer_hbm, m_v.at[0])
    for i in range(RPT // CH):
        pltpu.sync_copy(m_v.at[0], acc_sh.at[pl.ds(s * RPT + i * CH, CH)])
    pltpu.sync_copy(dst_hbm.at[pl.ds(wid * EPW_CH, EPW_CH)], idx_v)
    plsc.subcore_barrier()

    def issue(j, b):
        pltpu.async_copy(msg_hbm.at[pl.ds(base + j * CH, CH)], m_v.at[b], sg[b])

    def s_wait(b):
        pltpu.make_async_copy(
            m_v.at[b], acc_sh.at[pl.ds(0, CH)], ss[b]).wait()

    issue(0, 0)

    def pair(jj, carry):
        for b in range(2):
            j = jj * 2 + b
            off = base + j * CH
            pltpu.make_async_copy(
                msg_hbm.at[pl.ds(off, CH)], m_v.at[b], sg[b]).wait()
            pltpu.async_copy(m_v.at[b], acc_sh.at[idx_v.at[j]], ss[b], add=True)
            j2 = j + 1
            b2 = (b + 1) % 2

            @pl.when(j2 < EPW_CH)
            def _():
                @pl.when(j2 >= 2)
                def _():
                    s_wait(b2)
                issue(j2, b2)
        return carry

    lax.fori_loop(0, EPW_CH // 2, pair, 0)
    s_wait(0)
    s_wait(1)
    plsc.subcore_barrier()

    for i in range(RPT // CH):
        r0 = s * RPT + i * CH
        pltpu.sync_copy(acc_sh.at[pl.ds(r0, CH)], m_v.at[0])
        pltpu.sync_copy(m_v.at[0], out_hbm.at[c, pl.ds(r0, CH)])


def _sc_scatter(msg, dstp, zer):
    mesh = plsc.VectorSubcoreMesh(core_axis_name="c", subcore_axis_name="s")
    f = pl.kernel(
        _scatter_body,
        out_type=jax.ShapeDtypeStruct((2, NACC, LD), jnp.float32),
        mesh=mesh,
        scratch_types=[
            pltpu.VMEM((EPW_CH, CH), jnp.int32),
            pltpu.VMEM((2, CH, LD), jnp.float32),
            pltpu.VMEM_SHARED((NACC, LD), jnp.float32),
            pltpu.SemaphoreType.DMA,
            pltpu.SemaphoreType.DMA,
            pltpu.SemaphoreType.DMA,
            pltpu.SemaphoreType.DMA,
        ],
    )
    return f(msg, dstp, zer)


RT = 2000
TE = 4096


def _ln(t, g, b):
    mu = jnp.mean(t, axis=-1, keepdims=True)
    var = jnp.mean((t - mu) ** 2, axis=-1, keepdims=True)
    return (t - mu) * lax.rsqrt(var + LN_EPS) * g + b


def _dot(a, w):
    return jnp.dot(a, w, preferred_element_type=jnp.float32)


def _enc_body(nin_ref, tar_ref,
              w0, b0, w1, b1, w2, b2, g, be, w1s, w1d, bh,
              pos_ref, x_ref, a_ref, bv_ref):
    tp = nin_ref[:, 4:5]
    preset = jnp.logical_and(tp != 0.0, tp != 5.0)
    pos = jnp.where(preset, tar_ref[...], nin_ref[:, 0:2])
    lat = jnp.concatenate([pos, tp], axis=1)
    h = jnp.maximum(_dot(lat, w0[...]) + b0[...], 0.0)
    h = jnp.maximum(_dot(h, w1[...]) + b1[...], 0.0)
    h = _dot(h, w2[...]) + b2[...]
    x = _ln(h, g[...], be[...])
    pos_ref[...] = pos
    x_ref[...] = x
    a_ref[...] = _dot(x, w1s[...])
    bv_ref[...] = _dot(x, w1d[...]) + bh[...]


def _edge_body(ha_ref, hb_ref, w2, b2, w3, b3, g, be, o_ref):
    t = jnp.maximum(ha_ref[...] + hb_ref[...], 0.0)
    t = jnp.maximum(_dot(t, w2[...]) + b2[...], 0.0)
    t = _dot(t, w3[...]) + b3[...]
    o_ref[...] = _ln(t, g[...], be[...])


def _node_body(x_ref, a0_ref, a1_ref,
               v1x, v1a, c1, v2, c2, v3, c3, g, be, w1s, w1d, bh,
               xn_ref, a_ref, bv_ref):
    x = x_ref[...]
    agg = a0_ref[...] + a1_ref[...]
    u = jnp.maximum(_dot(x, v1x[...]) + _dot(agg, v1a[...]) + c1[...], 0.0)
    u = jnp.maximum(_dot(u, v2[...]) + c2[...], 0.0)
    u = _dot(u, v3[...]) + c3[...]
    xn = x + _ln(u, g[...], be[...])
    xn_ref[...] = xn
    a_ref[...] = _dot(xn, w1s[...])
    bv_ref[...] = _dot(xn, w1d[...]) + bh[...]


def _final_body(x_ref, a0_ref, a1_ref, nin_ref, tar_ref, pos_ref,
                v1x, v1a, c1, v2, c2, v3, c3, g, be,
                d0, e0, d1, e1, d2, e2,
                out_ref, loss_ref, nz_ref):
    i = pl.program_id(0)
    x = x_ref[...]
    agg = a0_ref[...] + a1_ref[...]
    u = jnp.maximum(_dot(x, v1x[...]) + _dot(agg, v1a[...]) + c1[...], 0.0)
    u = jnp.maximum(_dot(u, v2[...]) + c2[...], 0.0)
    u = _dot(u, v3[...]) + c3[...]
    xn = x + _ln(u, g[...], be[...])
    d = jnp.maximum(_dot(xn, d0[...]) + e0[...], 0.0)
    d = jnp.maximum(_dot(d, d1[...]) + e1[...], 0.0)
    d = _dot(d, d2[...]) + e2[...]
    outp = d + pos_ref[...]
    tp = nin_ref[:, 4:5]
    measure = jnp.logical_or(tp == 0.0, tp == 5.0)
    tar = tar_ref[...]
    out = jnp.where(measure, outp, tar)
    out_ref[...] = out
    diff2 = jnp.where(measure, (outp - tar) ** 2, 0.0)
    cnt = jnp.where(measure, jnp.float32(2.0), 0.0)

    @pl.when(i == 0)
    def _():
        loss_ref[...] = jnp.zeros((1, 1), jnp.float32)
        nz_ref[...] = jnp.zeros((1, 1), jnp.float32)

    loss_ref[...] += jnp.sum(diff2)[None, None]
    nz_ref[...] += jnp.sum(cnt)[None, None]


def _full(shape_fn=None):
    return pl.BlockSpec(shape_fn, lambda i: (0, 0))


def _tc_encode(nin, tar, wts):
    grid = (N // RT,)
    row = lambda shp: pl.BlockSpec(shp, lambda i: (i, 0))
    specs = [row((RT, 5)), row((RT, 2))]
    specs += [_full(w.shape) for w in wts]
    return pl.pallas_call(
        _enc_body,
        grid=grid,
        in_specs=specs,
        out_specs=[row((RT, 2)), row((RT, LD)), row((RT, LD)), row((RT, LD))],
        out_shape=[jax.ShapeDtypeStruct((N, 2), jnp.float32)] +
                  [jax.ShapeDtypeStruct((N, LD), jnp.float32)] * 3,
    )(nin, tar, *wts)


def _tc_edge(ha, hb, wts):
    grid = (EP // TE,)
    row = lambda shp: pl.BlockSpec(shp, lambda i: (i, 0))
    specs = [row((TE, LD)), row((TE, LD))]
    specs += [_full(w.shape) for w in wts]
    return pl.pallas_call(
        _edge_body,
        grid=grid,
        in_specs=specs,
        out_specs=row((TE, LD)),
        out_shape=jax.ShapeDtypeStruct((EP, LD), jnp.float32),
    )(ha, hb, *wts)


def _tc_node(x, agg0, agg1, wts):
    grid = (N // RT,)
    row = lambda shp: pl.BlockSpec(shp, lambda i: (i, 0))
    specs = [row((RT, LD))] * 3
    specs += [_full(w.shape) for w in wts]
    return pl.pallas_call(
        _node_body,
        grid=grid,
        in_specs=specs,
        out_specs=[row((RT, LD))] * 3,
        out_shape=[jax.ShapeDtypeStruct((N, LD), jnp.float32)] * 3,
    )(x, agg0, agg1, *wts)


def _tc_final(x, agg0, agg1, nin, tar, pos, wts):
    grid = (N // RT,)
    row = lambda shp: pl.BlockSpec(shp, lambda i: (i, 0))
    specs = [row((RT, LD))] * 3 + [row((RT, 5)), row((RT, 2)), row((RT, 2))]
    specs += [_full(w.shape) for w in wts]
    return pl.pallas_call(
        _final_body,
        grid=grid,
        in_specs=specs,
        out_specs=[row((RT, 2)), _full((1, 1)), _full((1, 1))],
        out_shape=[jax.ShapeDtypeStruct((N, 2), jnp.float32),
                   jax.ShapeDtypeStruct((1, 1), jnp.float32),
                   jax.ShapeDtypeStruct((1, 1), jnp.float32)],
    )(x, agg0, agg1, nin, tar, pos, *wts)


def _row(v):
    return v.reshape(1, -1)


def kernel(m_idx, m_gs, node_in, node_tar, params):
    del m_idx
    nin = node_in[0]
    tar = node_tar[0]
    src = m_gs[0].astype(jnp.int32)
    dst = m_gs[1].astype(jnp.int32)

    pad = EP - E
    srcp = jnp.concatenate(
        [src, jnp.zeros((pad,), jnp.int32)]).reshape(NW * EPW_CH, CH)
    dstp_g = jnp.concatenate(
        [dst, jnp.zeros((pad,), jnp.int32)]).reshape(NW * EPW_CH, CH)
    dstp_s = jnp.concatenate(
        [dst, jnp.full((pad,), N, jnp.int32)]).reshape(NW * EPW_CH, CH)
    zer = jnp.zeros((CH, LD), jnp.float32)

    enc = params["encode"]["layers"]
    enc_g, enc_b = params["encode"]["ln"]

    def edge_w(t):
        blk = params["gn"][t % LAYER_NUM]["edge"]
        (w1, b1), (w2, b2), (w3, b3) = blk["layers"]
        g, be = blk["ln"]
        return (w1[:LD], w1[LD:], _row(b1), w2, _row(b2), w3, _row(b3),
                _row(g), _row(be))

    def node_w(t):
        blk = params["gn"][t % LAYER_NUM]["node"]
        (v1, c1), (v2, c2), (v3, c3) = blk["layers"]
        g, be = blk["ln"]
        return (v1[:LD], v1[LD:], _row(c1), v2, _row(c2), v3, _row(c3),
                _row(g), _row(be))

    ew = [edge_w(t) for t in range(4)]
    nw = [node_w(t) for t in range(4)]
    dec = params["decode"]["layers"]
    dec_w = []
    for (w, b) in dec:
        dec_w += [w, _row(b)]

    enc_wts = [enc[0][0], _row(enc[0][1]), enc[1][0], _row(enc[1][1]),
               enc[2][0], _row(enc[2][1]), _row(enc_g), _row(enc_b),
               ew[0][0], ew[0][1], ew[0][2]]
    pos, x, a, bv = _tc_encode(nin, tar, enc_wts)

    for t in range(4):
        ha, hb = _sc_gather(a, bv, srcp, dstp_g)
        msg = _tc_edge(ha, hb, list(ew[t][3:]))
        part = _sc_scatter(msg, dstp_s, zer)
        agg0 = part[0, :N]
        agg1 = part[1, :N]
        if t < 3:
            nwts = list(nw[t]) + [ew[t + 1][0], ew[t + 1][1], ew[t + 1][2]]
            x, a, bv = _tc_node(x, agg0, agg1, nwts)
        else:
            fwts = list(nw[t]) + dec_w
            out, loss, nz = _tc_final(x, agg0, agg1, nin, tar, pos, fwts)

    nzs = nz[0, 0]
    return (loss[0, 0] / nzs, out[None], nzs)


# speedup vs baseline: 1.0228x; 1.0228x over previous
import functools

import jax
import jax.numpy as jnp
from jax import lax
from jax.experimental import pallas as pl
from jax.experimental.pallas import tpu as pltpu
from jax.experimental.pallas import tpu_sc as plsc

PD = 2
LD = 128
N = 10000
E = 320000
MP_TIMES = 2
LAYER_NUM = 2

NW = 32
CH = 128
EPW_CH = 80
EPW = CH * EPW_CH
EP = EPW * NW
NACC = 10240
RPT = NACC // 16
LN_EPS = 1e-5


def _gather_body(a_hbm, b_hbm, src_hbm, dst_hbm, ha_hbm, hb_hbm,
                 is_v, id_v, ra_v, rb_v,
                 sga0, sga1, sgb0, sgb1, swa0, swa1, swb0, swb1):
    wid = lax.axis_index("s") * 2 + lax.axis_index("c")
    base = wid * EPW
    sga = (sga0, sga1)
    sgb = (sgb0, sgb1)
    swa = (swa0, swa1)
    swb = (swb0, swb1)

    pltpu.sync_copy(src_hbm.at[pl.ds(wid * EPW_CH, EPW_CH)], is_v)
    pltpu.sync_copy(dst_hbm.at[pl.ds(wid * EPW_CH, EPW_CH)], id_v)

    def issue(j, b):
        pltpu.async_copy(a_hbm.at[is_v.at[j]], ra_v.at[b], sga[b])
        pltpu.async_copy(b_hbm.at[id_v.at[j]], rb_v.at[b], sgb[b])

    issue(0, 0)
    issue(1, 1)

    def pair(jj, carry):
        for b in range(2):
            j = jj * 2 + b
            off = base + j * CH
            pltpu.make_async_copy(a_hbm.at[is_v.at[j]], ra_v.at[b], sga[b]).wait()
            pltpu.make_async_copy(b_hbm.at[id_v.at[j]], rb_v.at[b], sgb[b]).wait()
            ca = pltpu.async_copy(ra_v.at[b], ha_hbm.at[pl.ds(off, CH)], swa[b])
            cb = pltpu.async_copy(rb_v.at[b], hb_hbm.at[pl.ds(off, CH)], swb[b])
            ca.wait()
            cb.wait()

            @pl.when(j + 2 < EPW_CH)
            def _():
                issue(j + 2, b)
        return carry

    lax.fori_loop(0, EPW_CH // 2, pair, 0)


def _sc_gather(a, b, srcp, dstp):
    mesh = plsc.VectorSubcoreMesh(core_axis_name="c", subcore_axis_name="s")
    f = pl.kernel(
        _gather_body,
        out_type=(jax.ShapeDtypeStruct((EP, LD), jnp.float32),
                  jax.ShapeDtypeStruct((EP, LD), jnp.float32)),
        mesh=mesh,
        scratch_types=[
            pltpu.VMEM((EPW_CH, CH), jnp.int32),
            pltpu.VMEM((EPW_CH, CH), jnp.int32),
            pltpu.VMEM((2, CH, LD), jnp.float32),
            pltpu.VMEM((2, CH, LD), jnp.float32),
        ] + [pltpu.SemaphoreType.DMA] * 8,
    )
    return f(a, b, srcp, dstp)


def _scatter_body(msg_hbm, dst_hbm, zer_hbm, out_hbm,
                  idx_v, m_v, acc_sh, sg0, sg1):
    c = lax.axis_index("c")
    s = lax.axis_index("s")
    wid = s * 2 + c
    base = wid * EPW
    sg = (sg0, sg1)

    pltpu.sync_copy(zer_hbm, m_v.at[0])
    for i in range(RPT // CH):
        pltpu.sync_copy(m_v.at[0], acc_sh.at[pl.ds(s * RPT + i * CH, CH)])
    pltpu.sync_copy(dst_hbm.at[pl.ds(wid * EPW_CH, EPW_CH)], idx_v)
    plsc.subcore_barrier()

    def issue(j, b):
        pltpu.async_copy(msg_hbm.at[pl.ds(base + j * CH, CH)], m_v.at[b], sg[b])

    issue(0, 0)
    issue(1, 1)

    def pair(jj, carry):
        for b in range(2):
            j = jj * 2 + b
            off = base + j * CH
            pltpu.make_async_copy(
                msg_hbm.at[pl.ds(off, CH)], m_v.at[b], sg[b]).wait()
            pltpu.sync_copy(m_v.at[b], acc_sh.at[idx_v.at[j]], add=True)

            @pl.when(j + 2 < EPW_CH)
            def _():
                issue(j + 2, b)
        return carry

    lax.fori_loop(0, EPW_CH // 2, pair, 0)
    plsc.subcore_barrier()

    for i in range(RPT // CH):
        r0 = s * RPT + i * CH
        pltpu.sync_copy(acc_sh.at[pl.ds(r0, CH)], m_v.at[0])
        pltpu.sync_copy(m_v.at[0], out_hbm.at[c, pl.ds(r0, CH)])


def _sc_scatter(msg, dstp, zer):
    mesh = plsc.VectorSubcoreMesh(core_axis_name="c", subcore_axis_name="s")
    f = pl.kernel(
        _scatter_body,
        out_type=jax.ShapeDtypeStruct((2, NACC, LD), jnp.float32),
        mesh=mesh,
        scratch_types=[
            pltpu.VMEM((EPW_CH, CH), jnp.int32),
            pltpu.VMEM((2, CH, LD), jnp.float32),
            pltpu.VMEM_SHARED((NACC, LD), jnp.float32),
            pltpu.SemaphoreType.DMA,
            pltpu.SemaphoreType.DMA,
        ],
    )
    return f(msg, dstp, zer)


RT = 2000
TE = 4096


def _ln(t, g, b):
    mu = jnp.mean(t, axis=-1, keepdims=True)
    var = jnp.mean((t - mu) ** 2, axis=-1, keepdims=True)
    return (t - mu) * lax.rsqrt(var + LN_EPS) * g + b


def _dot(a, w):
    return jnp.dot(a, w, preferred_element_type=jnp.float32)


def _enc_body(nin_ref, tar_ref,
              w0, b0, w1, b1, w2, b2, g, be, w1s, w1d, bh,
              pos_ref, x_ref, a_ref, bv_ref):
    tp = nin_ref[:, 4:5]
    preset = jnp.logical_and(tp != 0.0, tp != 5.0)
    pos = jnp.where(preset, tar_ref[...], nin_ref[:, 0:2])
    lat = jnp.concatenate([pos, tp], axis=1)
    h = jnp.maximum(_dot(lat, w0[...]) + b0[...], 0.0)
    h = jnp.maximum(_dot(h, w1[...]) + b1[...], 0.0)
    h = _dot(h, w2[...]) + b2[...]
    x = _ln(h, g[...], be[...])
    pos_ref[...] = pos
    x_ref[...] = x
    a_ref[...] = _dot(x, w1s[...])
    bv_ref[...] = _dot(x, w1d[...]) + bh[...]


def _edge_body(ha_ref, hb_ref, w2, b2, w3, b3, g, be, o_ref):
    t = jnp.maximum(ha_ref[...] + hb_ref[...], 0.0)
    t = jnp.maximum(_dot(t, w2[...]) + b2[...], 0.0)
    t = _dot(t, w3[...]) + b3[...]
    o_ref[...] = _ln(t, g[...], be[...])


def _node_body(x_ref, a0_ref, a1_ref,
               v1x, v1a, c1, v2, c2, v3, c3, g, be, w1s, w1d, bh,
               xn_ref, a_ref, bv_ref):
    x = x_ref[...]
    agg = a0_ref[...] + a1_ref[...]
    u = jnp.maximum(_dot(x, v1x[...]) + _dot(agg, v1a[...]) + c1[...], 0.0)
    u = jnp.maximum(_dot(u, v2[...]) + c2[...], 0.0)
    u = _dot(u, v3[...]) + c3[...]
    xn = x + _ln(u, g[...], be[...])
    xn_ref[...] = xn
    a_ref[...] = _dot(xn, w1s[...])
    bv_ref[...] = _dot(xn, w1d[...]) + bh[...]


def _final_body(x_ref, a0_ref, a1_ref, nin_ref, tar_ref, pos_ref,
                v1x, v1a, c1, v2, c2, v3, c3, g, be,
                d0, e0, d1, e1, d2, e2,
                out_ref, loss_ref, nz_ref):
    i = pl.program_id(0)
    x = x_ref[...]
    agg = a0_ref[...] + a1_ref[...]
    u = jnp.maximum(_dot(x, v1x[...]) + _dot(agg, v1a[...]) + c1[...], 0.0)
    u = jnp.maximum(_dot(u, v2[...]) + c2[...], 0.0)
    u = _dot(u, v3[...]) + c3[...]
    xn = x + _ln(u, g[...], be[...])
    d = jnp.maximum(_dot(xn, d0[...]) + e0[...], 0.0)
    d = jnp.maximum(_dot(d, d1[...]) + e1[...], 0.0)
    d = _dot(d, d2[...]) + e2[...]
    outp = d + pos_ref[...]
    tp = nin_ref[:, 4:5]
    measure = jnp.logical_or(tp == 0.0, tp == 5.0)
    tar = tar_ref[...]
    out = jnp.where(measure, outp, tar)
    out_ref[...] = out
    diff2 = jnp.where(measure, (outp - tar) ** 2, 0.0)
    cnt = jnp.where(measure, jnp.float32(2.0), 0.0)

    @pl.when(i == 0)
    def _():
        loss_ref[...] = jnp.zeros((1, 1), jnp.float32)
        nz_ref[...] = jnp.zeros((1, 1), jnp.float32)

    loss_ref[...] += jnp.sum(diff2)[None, None]
    nz_ref[...] += jnp.sum(cnt)[None, None]


def _full(shape_fn=None):
    return pl.BlockSpec(shape_fn, lambda i: (0, 0))


def _tc_encode(nin, tar, wts):
    grid = (N // RT,)
    row = lambda shp: pl.BlockSpec(shp, lambda i: (i, 0))
    specs = [row((RT, 5)), row((RT, 2))]
    specs += [_full(w.shape) for w in wts]
    return pl.pallas_call(
        _enc_body,
        grid=grid,
        in_specs=specs,
        out_specs=[row((RT, 2)), row((RT, LD)), row((RT, LD)), row((RT, LD))],
        out_shape=[jax.ShapeDtypeStruct((N, 2), jnp.float32)] +
                  [jax.ShapeDtypeStruct((N, LD), jnp.float32)] * 3,
    )(nin, tar, *wts)


def _tc_edge(ha, hb, wts):
    grid = (EP // TE,)
    row = lambda shp: pl.BlockSpec(shp, lambda i: (i, 0))
    specs = [row((TE, LD)), row((TE, LD))]
    specs += [_full(w.shape) for w in wts]
    return pl.pallas_call(
        _edge_body,
        grid=grid,
        in_specs=specs,
        out_specs=row((TE, LD)),
        out_shape=jax.ShapeDtypeStruct((EP, LD), jnp.float32),
    )(ha, hb, *wts)


def _tc_node(x, agg0, agg1, wts):
    grid = (N // RT,)
    row = lambda shp: pl.BlockSpec(shp, lambda i: (i, 0))
    specs = [row((RT, LD))] * 3
    specs += [_full(w.shape) for w in wts]
    return pl.pallas_call(
        _node_body,
        grid=grid,
        in_specs=specs,
        out_specs=[row((RT, LD))] * 3,
        out_shape=[jax.ShapeDtypeStruct((N, LD), jnp.float32)] * 3,
    )(x, agg0, agg1, *wts)


def _tc_final(x, agg0, agg1, nin, tar, pos, wts):
    grid = (N // RT,)
    row = lambda shp: pl.BlockSpec(shp, lambda i: (i, 0))
    specs = [row((RT, LD))] * 3 + [row((RT, 5)), row((RT, 2)), row((RT, 2))]
    specs += [_full(w.shape) for w in wts]
    return pl.pallas_call(
        _final_body,
        grid=grid,
        in_specs=specs,
        out_specs=[row((RT, 2)), _full((1, 1)), _full((1, 1))],
        out_shape=[jax.ShapeDtypeStruct((N, 2), jnp.float32),
                   jax.ShapeDtypeStruct((1, 1), jnp.float32),
                   jax.ShapeDtypeStruct((1, 1), jnp.float32)],
    )(x, agg0, agg1, nin, tar, pos, *wts)


def _row(v):
    return v.reshape(1, -1)


def kernel(m_idx, m_gs, node_in, node_tar, params):
    del m_idx
    nin = node_in[0]
    tar = node_tar[0]
    src = m_gs[0].astype(jnp.int32)
    dst = m_gs[1].astype(jnp.int32)

    pad = EP - E
    srcp = jnp.concatenate(
        [src, jnp.zeros((pad,), jnp.int32)]).reshape(NW * EPW_CH, CH)
    dstp_g = jnp.concatenate(
        [dst, jnp.zeros((pad,), jnp.int32)]).reshape(NW * EPW_CH, CH)
    dstp_s = jnp.concatenate(
        [dst, jnp.full((pad,), N, jnp.int32)]).reshape(NW * EPW_CH, CH)
    zer = jnp.zeros((CH, LD), jnp.float32)

    enc = params["encode"]["layers"]
    enc_g, enc_b = params["encode"]["ln"]

    def edge_w(t):
        blk = params["gn"][t % LAYER_NUM]["edge"]
        (w1, b1), (w2, b2), (w3, b3) = blk["layers"]
        g, be = blk["ln"]
        return (w1[:LD], w1[LD:], _row(b1), w2, _row(b2), w3, _row(b3),
                _row(g), _row(be))

    def node_w(t):
        blk = params["gn"][t % LAYER_NUM]["node"]
        (v1, c1), (v2, c2), (v3, c3) = blk["layers"]
        g, be = blk["ln"]
        return (v1[:LD], v1[LD:], _row(c1), v2, _row(c2), v3, _row(c3),
                _row(g), _row(be))

    ew = [edge_w(t) for t in range(4)]
    nw = [node_w(t) for t in range(4)]
    dec = params["decode"]["layers"]
    dec_w = []
    for (w, b) in dec:
        dec_w += [w, _row(b)]

    enc_wts = [enc[0][0], _row(enc[0][1]), enc[1][0], _row(enc[1][1]),
               enc[2][0], _row(enc[2][1]), _row(enc_g), _row(enc_b),
               ew[0][0], ew[0][1], ew[0][2]]
    pos, x, a, bv = _tc_encode(nin, tar, enc_wts)

    for t in range(4):
        ha, hb = _sc_gather(a, bv, srcp, dstp_g)
        msg = _tc_edge(ha, hb, list(ew[t][3:]))
        part = _sc_scatter(msg, dstp_s, zer)
        agg0 = part[0, :N]
        agg1 = part[1, :N]
        if t < 3:
            nwts = list(nw[t]) + [ew[t + 1][0], ew[t + 1][1], ew[t + 1][2]]
            x, a, bv = _tc_node(x, agg0, agg1, nwts)
        else:
            fwts = list(nw[t]) + dec_w
            out, loss, nz = _tc_final(x, agg0, agg1, nin, tar, pos, fwts)

    nzs = nz[0, 0]
    return (loss[0, 0] / nzs, out[None], nzs)
